# baseline (device time: 20205 ns/iter reference)
import jax
import jax.numpy as jnp
from jax import lax
from jax.experimental import pallas as pl
from jax.experimental.pallas import tpu as pltpu

_CQ = 16
_HF = _CQ // 2


def kernel(x):
    m, n = x.shape
    qrows = m // 4
    rpc = qrows // _CQ

    def body(x_ref, out_ref, zbuf, z_send, z_recv, x_send, x_recv,
             y_send, y_recv):
        mx = lax.axis_index("x")
        my = lax.axis_index("y")
        mz = lax.axis_index("z")
        q = 2 * my + mx
        qx = 2 * my + (1 - mx)
        qy = 2 * (1 - my) + mx
        zp = (mx, my, 1 - mz)
        xp = (1 - mx, my, mz)
        yp = (mx, 1 - my, mz)

        barrier_sem = pltpu.get_barrier_semaphore()
        for nbr in (zp, xp, yp):
            pl.semaphore_signal(
                barrier_sem, inc=1,
                device_id=nbr, device_id_type=pl.DeviceIdType.MESH,
            )
        pl.semaphore_wait(barrier_sem, 3)

        row0 = q * qrows
        rowx = qx * qrows
        rowy = qy * qrows

        def copy(rows, send_sem, recv_sem, dev):
            return pltpu.make_async_remote_copy(
                src_ref=out_ref.at[pl.ds(rows, rpc), :],
                dst_ref=out_ref.at[pl.ds(rows, rpc), :],
                send_sem=send_sem,
                recv_sem=recv_sem,
                device_id=dev,
                device_id_type=pl.DeviceIdType.MESH,
            )

        z_rdmas = []
        for c in range(_CQ):
            r = pltpu.make_async_remote_copy(
                src_ref=x_ref.at[pl.ds(row0 + c * rpc, rpc), :],
                dst_ref=zbuf.at[pl.ds(c * rpc, rpc), :],
                send_sem=z_send.at[c],
                recv_sem=z_recv.at[c],
                device_id=zp,
                device_id_type=pl.DeviceIdType.MESH,
            )
            r.start()
            z_rdmas.append(r)

        xq = []
        yq = []
        for c in range(_CQ):
            z_rdmas[c].wait_recv()
            out_ref[pl.ds(row0 + c * rpc, rpc), :] = (
                x_ref[pl.ds(row0 + c * rpc, rpc), :]
                + zbuf[pl.ds(c * rpc, rpc), :]
            )
            rx = copy(row0 + c * rpc, x_send.at[c], x_recv.at[c], xp)
            rx.start()
            xq.append(rx)
            ry = copy(row0 + c * rpc, y_send.at[c], y_recv.at[c], yp)
            ry.start()
            yq.append(ry)

        xf = []
        yf = []
        for c in range(_HF):
            yq[c].wait_recv()
            rf = copy(rowy + c * rpc, x_send.at[_CQ + c],
                      x_recv.at[_CQ + c], xp)
            rf.start()
            xf.append(rf)
            xq[_HF + c].wait_recv()
            rf = copy(rowx + (_HF + c) * rpc, y_send.at[_CQ + c],
                      y_recv.at[_CQ + c], yp)
            rf.start()
            yf.append(rf)

        for c in range(_HF):
            xq[c].wait_recv()
            yq[_HF + c].wait_recv()
        for r in xf + yf:
            r.wait_recv()
        for r in z_rdmas + xq + yq + xf + yf:
            r.wait_send()

    return pl.pallas_call(
        body,
        out_shape=jax.ShapeDtypeStruct((m, n), x.dtype),
        in_specs=[pl.BlockSpec(memory_space=pltpu.VMEM)],
        out_specs=pl.BlockSpec(memory_space=pltpu.VMEM),
        scratch_shapes=[
            pltpu.VMEM((qrows, n), x.dtype),
            pltpu.SemaphoreType.DMA((_CQ,)),
            pltpu.SemaphoreType.DMA((_CQ,)),
            pltpu.SemaphoreType.DMA((_CQ + _HF,)),
            pltpu.SemaphoreType.DMA((_CQ + _HF,)),
            pltpu.SemaphoreType.DMA((_CQ + _HF,)),
            pltpu.SemaphoreType.DMA((_CQ + _HF,)),
        ],
        compiler_params=pltpu.CompilerParams(collective_id=0),
    )(x)


# device time: 20043 ns/iter; 1.0081x vs baseline; 1.0081x over previous
import functools

import jax
import jax.numpy as jnp
from jax import lax
from jax.experimental import pallas as pl
from jax.experimental.pallas import tpu as pltpu

_CQ = 8
_HF = _CQ // 2


def kernel(x):
    m, n = x.shape
    qrows = m // 4
    rpc = qrows // _CQ

    def body(x_ref, out_ref, zbuf, z_send, z_recv, x_send, x_recv,
             y_send, y_recv):
        mx = lax.axis_index("x")
        my = lax.axis_index("y")
        mz = lax.axis_index("z")
        q = 2 * my + mx
        qx = 2 * my + (1 - mx)
        qy = 2 * (1 - my) + mx
        zp = (mx, my, 1 - mz)
        xp = (1 - mx, my, mz)
        yp = (mx, 1 - my, mz)

        skip_x = 2 * mz
        skip_y = 4 + mz
        fwd_x = 2 * (1 - mz)
        fwd_y = 5 - mz

        barrier_sem = pltpu.get_barrier_semaphore()
        for nbr in (zp, xp, yp):
            pl.semaphore_signal(
                barrier_sem, inc=1,
                device_id=nbr, device_id_type=pl.DeviceIdType.MESH,
            )
        pl.semaphore_wait(barrier_sem, 3)

        row0 = q * qrows
        rowx = qx * qrows
        rowy = qy * qrows

        def copy(rows, send_sem, recv_sem, dev):
            return pltpu.make_async_remote_copy(
                src_ref=out_ref.at[pl.ds(rows, rpc), :],
                dst_ref=out_ref.at[pl.ds(rows, rpc), :],
                send_sem=send_sem,
                recv_sem=recv_sem,
                device_id=dev,
                device_id_type=pl.DeviceIdType.MESH,
            )

        z_rdmas = []
        for c in range(_CQ):
            r = pltpu.make_async_remote_copy(
                src_ref=x_ref.at[pl.ds(row0 + c * rpc, rpc), :],
                dst_ref=zbuf.at[pl.ds(c * rpc, rpc), :],
                send_sem=z_send.at[c],
                recv_sem=z_recv.at[c],
                device_id=zp,
                device_id_type=pl.DeviceIdType.MESH,
            )
            r.start()
            z_rdmas.append(r)

        zf_x = copy(rowx + fwd_x * rpc, z_send.at[_CQ], z_recv.at[_CQ], zp)
        zf_y = copy(rowy + fwd_y * rpc, z_send.at[_CQ + 1],
                    z_recv.at[_CQ + 1], zp)

        xq = []
        yq = []
        for c in range(_CQ):
            z_rdmas[c].wait_recv()
            out_ref[pl.ds(row0 + c * rpc, rpc), :] = (
                x_ref[pl.ds(row0 + c * rpc, rpc), :]
                + zbuf[pl.ds(c * rpc, rpc), :]
            )
            rx = copy(row0 + c * rpc, x_send.at[c], x_recv.at[c], xp)
            xq.append(rx)

            @pl.when(c != skip_x)
            def _(rx=rx):
                rx.start()

            ry = copy(row0 + c * rpc, y_send.at[c], y_recv.at[c], yp)
            yq.append(ry)

            @pl.when(c != skip_y)
            def _(ry=ry):
                ry.start()

        for c in range(_CQ):
            @pl.when(c != skip_x)
            def _(c=c):
                xq[c].wait_recv()

            @pl.when(c == fwd_x)
            def _():
                zf_x.start()

            @pl.when(c != skip_y)
            def _(c=c):
                yq[c].wait_recv()

            @pl.when(c == fwd_y)
            def _():
                zf_y.start()

            if c < _HF:
                rf = copy(rowy + c * rpc, x_send.at[_CQ + 2 + c],
                          x_recv.at[_CQ + 2 + c], xp)
                rf.start()
                xq.append(rf)
            else:
                rf = copy(rowx + c * rpc, y_send.at[_CQ + 2 + c - _HF],
                          y_recv.at[_CQ + 2 + c - _HF], yp)
                rf.start()
                yq.append(rf)

        zf_x.wait_recv()
        zf_y.wait_recv()
        for r in xq[_CQ:] + yq[_CQ:]:
            r.wait_recv()
        for r in z_rdmas + [zf_x, zf_y] + xq[_CQ:] + yq[_CQ:]:
            r.wait_send()
        for c in range(_CQ):
            @pl.when(c != skip_x)
            def _(c=c):
                xq[c].wait_send()

            @pl.when(c != skip_y)
            def _(c=c):
                yq[c].wait_send()

    return pl.pallas_call(
        body,
        out_shape=jax.ShapeDtypeStruct((m, n), x.dtype),
        in_specs=[pl.BlockSpec(memory_space=pltpu.VMEM)],
        out_specs=pl.BlockSpec(memory_space=pltpu.VMEM),
        scratch_shapes=[
            pltpu.VMEM((qrows, n), x.dtype),
            pltpu.SemaphoreType.DMA((_CQ + 2,)),
            pltpu.SemaphoreType.DMA((_CQ + 2,)),
            pltpu.SemaphoreType.DMA((_CQ + 2 + _HF,)),
            pltpu.SemaphoreType.DMA((_CQ + 2 + _HF,)),
            pltpu.SemaphoreType.DMA((_CQ + 2 + _HF,)),
            pltpu.SemaphoreType.DMA((_CQ + 2 + _HF,)),
        ],
        compiler_params=pltpu.CompilerParams(collective_id=0),
    )(x)


# device time: 19937 ns/iter; 1.0134x vs baseline; 1.0053x over previous
import jax
import jax.numpy as jnp
from jax import lax
from jax.experimental import pallas as pl
from jax.experimental.pallas import tpu as pltpu

_CQ = 8
_HF = _CQ // 2


def kernel(x):
    m, n = x.shape
    qrows = m // 4
    rpc = qrows // _CQ

    def body(x_ref, out_ref, zbuf, z_send, z_recv, x_send, x_recv,
             y_send, y_recv):
        mx = lax.axis_index("x")
        my = lax.axis_index("y")
        mz = lax.axis_index("z")
        q = 2 * my + mx
        qx = 2 * my + (1 - mx)
        qy = 2 * (1 - my) + mx
        zp = (mx, my, 1 - mz)
        xp = (1 - mx, my, mz)
        yp = (mx, 1 - my, mz)

        barrier_sem = pltpu.get_barrier_semaphore()
        for nbr in (zp, xp, yp):
            pl.semaphore_signal(
                barrier_sem, inc=1,
                device_id=nbr, device_id_type=pl.DeviceIdType.MESH,
            )
        pl.semaphore_wait(barrier_sem, 3)

        row0 = q * qrows
        rowx = qx * qrows
        rowy = qy * qrows

        def copy(rows, send_sem, recv_sem, dev):
            return pltpu.make_async_remote_copy(
                src_ref=out_ref.at[pl.ds(rows, rpc), :],
                dst_ref=out_ref.at[pl.ds(rows, rpc), :],
                send_sem=send_sem,
                recv_sem=recv_sem,
                device_id=dev,
                device_id_type=pl.DeviceIdType.MESH,
            )

        z_rdmas = []
        for c in range(_CQ):
            r = pltpu.make_async_remote_copy(
                src_ref=x_ref.at[pl.ds(row0 + c * rpc, rpc), :],
                dst_ref=zbuf.at[pl.ds(c * rpc, rpc), :],
                send_sem=z_send.at[c],
                recv_sem=z_recv.at[c],
                device_id=zp,
                device_id_type=pl.DeviceIdType.MESH,
            )
            r.start()
            z_rdmas.append(r)

        xq = []
        yq = []
        for c in range(_CQ):
            z_rdmas[c].wait_recv()
            out_ref[pl.ds(row0 + c * rpc, rpc), :] = (
                x_ref[pl.ds(row0 + c * rpc, rpc), :]
                + zbuf[pl.ds(c * rpc, rpc), :]
            )
            rx = copy(row0 + c * rpc, x_send.at[c], x_recv.at[c], xp)
            rx.start()
            xq.append(rx)
            ry = copy(row0 + c * rpc, y_send.at[c], y_recv.at[c], yp)
            ry.start()
            yq.append(ry)

        xf = []
        yf = []
        for c in range(_HF):
            yq[c].wait_recv()
            rf = copy(rowy + c * rpc, x_send.at[_CQ + c],
                      x_recv.at[_CQ + c], xp)
            rf.start()
            xf.append(rf)
            xq[_HF + c].wait_recv()
            rf = copy(rowx + (_HF + c) * rpc, y_send.at[_CQ + c],
                      y_recv.at[_CQ + c], yp)
            rf.start()
            yf.append(rf)

        for c in range(_HF):
            xq[c].wait_recv()
            yq[_HF + c].wait_recv()
        for r in xf + yf:
            r.wait_recv()
        for r in z_rdmas + xq + yq + xf + yf:
            r.wait_send()

    return pl.pallas_call(
        body,
        out_shape=jax.ShapeDtypeStruct((m, n), x.dtype),
        in_specs=[pl.BlockSpec(memory_space=pltpu.VMEM)],
        out_specs=pl.BlockSpec(memory_space=pltpu.VMEM),
        scratch_shapes=[
            pltpu.VMEM((qrows, n), x.dtype),
            pltpu.SemaphoreType.DMA((_CQ,)),
            pltpu.SemaphoreType.DMA((_CQ,)),
            pltpu.SemaphoreType.DMA((_CQ + _HF,)),
            pltpu.SemaphoreType.DMA((_CQ + _HF,)),
            pltpu.SemaphoreType.DMA((_CQ + _HF,)),
            pltpu.SemaphoreType.DMA((_CQ + _HF,)),
        ],
        compiler_params=pltpu.CompilerParams(collective_id=0),
    )(x)
